# Initial kernel scaffold; baseline (speedup 1.0000x reference)
import jax
import jax.numpy as jnp
from jax.experimental import pallas as pl


def kernel(x, edge_index, W1, a_src1, a_dst1, Wl1, bl1, W2, a_src2, a_dst2, Wl2, bl2, W_lin1, b_lin1, W_lin2, b_lin2):
    raise NotImplementedError("write your pallas kernel here")


# speedup vs baseline: 23.3296x; 23.3296x over previous
import dataclasses
import functools

import jax
import jax.numpy as jnp
from jax import lax
from jax.experimental import pallas as pl
from jax.experimental.pallas import tpu as pltpu
from jax.experimental.pallas import tpu_sc as plsc

N = 10000
E = 320000
NFEAT = 128
NHID = 8
NHEADS = 4
LIN = 64
NCLASS = 10
ALPHA = 0.2

BN = 1000
NBN = N // BN
BN2 = 200
NBN2 = N // BN2
NWORKERS = 32


def _elu(v):
    return jnp.where(v > 0, v, jnp.exp(jnp.minimum(v, 0.0)) - 1.0)


def _leaky(v):
    return jnp.maximum(v, ALPHA * v)


def _prep1_body(x_ref, w1_ref, htab_ref):
    zpad = jnp.zeros((BN, NHID), jnp.float32)
    for h in range(NHEADS):
        hb = jnp.dot(x_ref[...], w1_ref[h],
                     preferred_element_type=jnp.float32)
        htab_ref[h] = jnp.concatenate([hb, zpad], axis=1)


def _prep1(x, W1):
    return pl.pallas_call(
        _prep1_body,
        grid=(NBN,),
        in_specs=[
            pl.BlockSpec((BN, NFEAT), lambda nb: (nb, 0)),
            pl.BlockSpec((NHEADS, NFEAT, NHID), lambda nb: (0, 0, 0)),
        ],
        out_specs=pl.BlockSpec((NHEADS, BN, 16), lambda nb: (0, nb, 0)),
        out_shape=jax.ShapeDtypeStruct((NHEADS, N, 16), jnp.float32),
    )(x, W1)


def _make_atten_scalars(NH):
    def body(htab_ref, as_ref, ad_ref, astab_ref, adtab_ref, btab_ref):
        hb = htab_ref[0, :, 0:NHID]
        asv = jnp.sum(hb * as_ref[0, 0][None, :], axis=1)
        adv = jnp.sum(hb * ad_ref[0, 0][None, :], axis=1)
        astab_ref[0, 0] = asv
        adtab_ref[0, 0] = adv
        b = _leaky(jnp.max(asv) + jnp.max(adv))
        btab_ref[...] = jnp.full((1, 1, 16), b, jnp.float32)

    def run(htab, a_src, a_dst):
        astab, adtab, btab = pl.pallas_call(
            body,
            grid=(NH,),
            in_specs=[
                pl.BlockSpec((1, N, 16), lambda h: (h, 0, 0)),
                pl.BlockSpec((1, 1, NHID), lambda h: (h, 0, 0)),
                pl.BlockSpec((1, 1, NHID), lambda h: (h, 0, 0)),
            ],
            out_specs=[
                pl.BlockSpec((1, 1, N), lambda h: (h, 0, 0)),
                pl.BlockSpec((1, 1, N), lambda h: (h, 0, 0)),
                pl.BlockSpec((1, 1, 16), lambda h: (h, 0, 0)),
            ],
            out_shape=[
                jax.ShapeDtypeStruct((NH, 1, N), jnp.float32),
                jax.ShapeDtypeStruct((NH, 1, N), jnp.float32),
                jax.ShapeDtypeStruct((NH, 1, 16), jnp.float32),
            ],
        )(htab, a_src.reshape(NH, 1, NHID), a_dst.reshape(NH, 1, NHID))
        return (astab.reshape(NH, N), adtab.reshape(NH, N),
                btab.reshape(NH, 16))
    return run


def _edge_pass_body(NH, NC, C,
                    htab, astab, adtab, btab, src_h, dst_h,
                    acc_o, den_o,
                    as_t, ad_t, bt, acc, den, srcb, dstb, gidxb, wbuf,
                    hrows, sem):
    EPW = E // NC
    NCH = EPW // C
    cid = lax.axis_index("c")
    sid = lax.axis_index("s")
    wid = sid * 2 + cid
    h = wid % NH
    c = wid // NH

    pltpu.sync_copy(astab.at[h], as_t)
    pltpu.sync_copy(adtab.at[h], ad_t)
    pltpu.sync_copy(btab.at[h], bt)

    zf = jnp.zeros((16,), jnp.float32)

    @pl.loop(0, N * NHID, step=16, unroll=8)
    def _(i):
        acc[pl.ds(i, 16)] = zf

    @pl.loop(0, N, step=16, unroll=8)
    def _(i):
        den[pl.ds(i, 16)] = zf

    bv = bt[...]
    iot = lax.iota(jnp.int32, 16)
    col8 = lax.rem(iot, 8)
    rep8 = lax.div(iot, 8)
    hoff = h * N

    @pl.loop(0, NCH)
    def _(k):
        base = c * EPW + k * C
        pltpu.sync_copy(src_h.at[pl.ds(base, C)], srcb)
        pltpu.sync_copy(dst_h.at[pl.ds(base, C)], dstb)

        @pl.loop(0, C, step=16, unroll=4)
        def _(i):
            gidxb[pl.ds(i, 16)] = srcb[pl.ds(i, 16)] + hoff

        pltpu.async_copy(htab.at[gidxb], hrows, sem).wait()

        @pl.loop(0, C, step=16, unroll=2)
        def _(i):
            sv = srcb[pl.ds(i, 16)]
            dv = dstb[pl.ds(i, 16)]
            asv = plsc.load_gather(as_t, [sv])
            adv = plsc.load_gather(ad_t, [dv])
            e = asv + adv
            e = jnp.maximum(e, ALPHA * e) - bv
            w = jnp.exp(e)
            wbuf[pl.ds(i, 16)] = w
            plsc.addupdate_scatter(den, [dv], w)

        @pl.loop(0, C, step=2, unroll=8)
        def _(i):
            iv = jnp.full((16,), i, jnp.int32) + rep8
            hv = plsc.load_gather(hrows, [iv, col8])
            wv = plsc.load_gather(wbuf, [iv])
            dv2 = plsc.load_gather(dstb, [iv])
            prod = hv * wv
            aidx = dv2 * 8 + col8
            plsc.addupdate_scatter(acc, [aidx], prod)

    pltpu.sync_copy(acc, acc_o.at[c, h])
    pltpu.sync_copy(den, den_o.at[c, h])


def _sc_compiler_params():
    cp = pltpu.CompilerParams()
    fields = pltpu.CompilerParams.__dataclass_fields__
    if "needs_layout_passes" in fields:
        cp = dataclasses.replace(cp, needs_layout_passes=False)
    if "use_tc_tiling_on_sc" in fields:
        cp = dataclasses.replace(cp, use_tc_tiling_on_sc=False)
    return cp


def _make_edge_pass(NH, NC, C):
    mesh = plsc.VectorSubcoreMesh(core_axis_name="c", subcore_axis_name="s")
    return pl.kernel(
        functools.partial(_edge_pass_body, NH, NC, C),
        out_type=[
            jax.ShapeDtypeStruct((NC, NH, N * NHID), jnp.float32),
            jax.ShapeDtypeStruct((NC, NH, N), jnp.float32),
        ],
        mesh=mesh,
        scratch_types=[
            pltpu.VMEM((N,), jnp.float32),
            pltpu.VMEM((N,), jnp.float32),
            pltpu.VMEM((16,), jnp.float32),
            pltpu.VMEM((N * NHID,), jnp.float32),
            pltpu.VMEM((N,), jnp.float32),
            pltpu.VMEM((C,), jnp.int32),
            pltpu.VMEM((C,), jnp.int32),
            pltpu.VMEM((C,), jnp.int32),
            pltpu.VMEM((C,), jnp.float32),
            pltpu.VMEM((C, 16), jnp.float32),
            pltpu.SemaphoreType.DMA,
        ],
        compiler_params=_sc_compiler_params(),
    )


def _prep2_body(acc_ref, den_ref, wl1_ref, bl1_ref, w2_ref, htab_ref):
    a = jnp.sum(acc_ref[...], axis=0)
    d = jnp.sum(den_ref[...], axis=0)[..., 0]
    zs = []
    for h in range(NHEADS):
        o = a[h] / (d[h][:, None] + 1e-16)
        o = _elu(o)
        z = _elu(jnp.dot(o, wl1_ref[h], preferred_element_type=jnp.float32)
                 + bl1_ref[h][None, :])
        zs.append(z)
    hcat = jnp.concatenate(zs, axis=1)
    h2 = jnp.dot(hcat, w2_ref[...], preferred_element_type=jnp.float32)
    htab_ref[...] = jnp.concatenate(
        [h2, jnp.zeros((BN2, NHID), jnp.float32)], axis=1)


def _prep2(acc1, den1, Wl1, bl1, W2):
    return pl.pallas_call(
        _prep2_body,
        grid=(NBN2,),
        in_specs=[
            pl.BlockSpec((8, NHEADS, BN2, NHID), lambda nb: (0, 0, nb, 0)),
            pl.BlockSpec((8, NHEADS, BN2, 1), lambda nb: (0, 0, nb, 0)),
            pl.BlockSpec((NHEADS, NHID, NHID), lambda nb: (0, 0, 0)),
            pl.BlockSpec((NHEADS, NHID), lambda nb: (0, 0)),
            pl.BlockSpec((NHEADS * NHID, NHID), lambda nb: (0, 0)),
        ],
        out_specs=pl.BlockSpec((BN2, 16), lambda nb: (nb, 0)),
        out_shape=jax.ShapeDtypeStruct((N, 16), jnp.float32),
    )(acc1, den1, Wl1, bl1, W2)


def _readout_body(acc_ref, den_ref, wl2_ref, bl2_ref, wlin1_ref, blin1_ref,
                  wlin2_ref, blin2_ref, out_ref, g_ref):
    nb = pl.program_id(0)
    o = jnp.sum(acc_ref[...], axis=0)
    d = jnp.sum(den_ref[...], axis=0)[:, 0]
    o = _elu(o / (d[:, None] + 1e-16))
    t = _elu(jnp.dot(o, wl2_ref[...], preferred_element_type=jnp.float32)
             + bl2_ref[...])
    t = _elu(t)
    t = _elu(jnp.dot(t, wlin1_ref[...], preferred_element_type=jnp.float32)
             + blin1_ref[...])
    gp = jnp.sum(t, axis=0)[None, :]

    @pl.when(nb == 0)
    def _():
        g_ref[...] = gp

    @pl.when(nb != 0)
    def _():
        g_ref[...] = g_ref[...] + gp

    @pl.when(nb == NBN2 - 1)
    def _():
        logits = (jnp.dot(g_ref[...], wlin2_ref[...],
                          preferred_element_type=jnp.float32)
                  + blin2_ref[...])
        m = jnp.max(logits)
        ls = logits - m
        out_ref[...] = ls - jnp.log(jnp.sum(jnp.exp(ls)))


def _readout(acc2, den2, Wl2, bl2, W_lin1, b_lin1, W_lin2, b_lin2):
    return pl.pallas_call(
        _readout_body,
        grid=(NBN2,),
        in_specs=[
            pl.BlockSpec((NWORKERS, BN2, NHID), lambda nb: (0, nb, 0)),
            pl.BlockSpec((NWORKERS, BN2, 1), lambda nb: (0, nb, 0)),
            pl.BlockSpec((NHID, LIN), lambda nb: (0, 0)),
            pl.BlockSpec((1, LIN), lambda nb: (0, 0)),
            pl.BlockSpec((LIN, 2 * LIN), lambda nb: (0, 0)),
            pl.BlockSpec((1, 2 * LIN), lambda nb: (0, 0)),
            pl.BlockSpec((2 * LIN, NCLASS), lambda nb: (0, 0)),
            pl.BlockSpec((1, NCLASS), lambda nb: (0, 0)),
        ],
        out_specs=[pl.BlockSpec((1, NCLASS), lambda nb: (0, 0))],
        out_shape=[jax.ShapeDtypeStruct((1, NCLASS), jnp.float32)],
        scratch_shapes=[pltpu.VMEM((1, 2 * LIN), jnp.float32)],
    )(acc2, den2, Wl2, bl2, W_lin1, b_lin1, W_lin2, b_lin2)


def kernel(x, edge_index, W1, a_src1, a_dst1, Wl1, bl1, W2, a_src2, a_dst2,
           Wl2, bl2, W_lin1, b_lin1, W_lin2, b_lin2):
    src = edge_index[0]
    dst = edge_index[1]

    htab1 = _prep1(x, W1)
    astab1, adtab1, btab1 = _make_atten_scalars(NHEADS)(htab1, a_src1, a_dst1)

    l1 = _make_edge_pass(NH=NHEADS, NC=8, C=400)
    acc1, den1 = l1(htab1.reshape(NHEADS * N, 16), astab1, adtab1, btab1,
                    src, dst)

    htab2 = _prep2(acc1.reshape(8, NHEADS, N, NHID),
                   den1.reshape(8, NHEADS, N, 1), Wl1, bl1, W2)
    astab2, adtab2, btab2 = _make_atten_scalars(1)(
        htab2.reshape(1, N, 16), a_src2.reshape(1, NHID),
        a_dst2.reshape(1, NHID))

    l2 = _make_edge_pass(NH=1, NC=NWORKERS, C=400)
    acc2, den2 = l2(htab2, astab2, adtab2, btab2, src, dst)

    (out,) = _readout(acc2.reshape(NWORKERS, N, NHID),
                      den2.reshape(NWORKERS, N, 1),
                      Wl2, bl2.reshape(1, LIN),
                      W_lin1, b_lin1.reshape(1, 2 * LIN),
                      W_lin2, b_lin2.reshape(1, NCLASS))
    return out


# speedup vs baseline: 41.5397x; 1.7806x over previous
import dataclasses
import functools

import jax
import jax.numpy as jnp
from jax import lax
from jax.experimental import pallas as pl
from jax.experimental.pallas import tpu as pltpu
from jax.experimental.pallas import tpu_sc as plsc

N = 10000
NP = 10240
E = 320000
NFEAT = 128
NHID = 8
NHEADS = 4
LIN = 64
NCLASS = 10
ALPHA = 0.2

BL = 1024
NBL = NP // BL
NWORKERS = 32


def _elu(v):
    return jnp.where(v > 0, v, jnp.exp(jnp.minimum(v, 0.0)) - 1.0)


def _leaky(v):
    return jnp.maximum(v, ALPHA * v)


def _dgt(a, b):
    return lax.dot_general(a, b, (((0,), (0,)), ((), ())),
                           preferred_element_type=jnp.float32)


def _prep1_body(x_ref, w1_ref, as1_ref, ad1_ref,
                htab_ref, astab_ref, adtab_ref, btab_ref, mx_ref):
    nb = pl.program_id(1)
    hh = pl.program_id(0)
    xb = x_ref[...]
    hb = jnp.dot(xb, w1_ref[0], preferred_element_type=jnp.float32)
    htab_ref[...] = jnp.concatenate(
        [hb, jnp.zeros((BL, NHID), jnp.float32)], axis=1)
    va_s = jnp.dot(w1_ref[0], as1_ref[0],
                   preferred_element_type=jnp.float32)
    va_d = jnp.dot(w1_ref[0], ad1_ref[0],
                   preferred_element_type=jnp.float32)
    asv = lax.dot_general(va_s, xb, (((0,), (1,)), ((), ())),
                          preferred_element_type=jnp.float32)
    adv = lax.dot_general(va_d, xb, (((0,), (1,)), ((), ())),
                          preferred_element_type=jnp.float32)
    astab_ref[...] = asv[None]
    adtab_ref[...] = adv[None]
    m_as = jnp.max(asv)
    m_ad = jnp.max(adv)

    @pl.when(nb == 0)
    def _():
        mx_ref[0] = m_as
        mx_ref[1] = m_ad

    @pl.when(nb != 0)
    def _():
        mx_ref[0] = jnp.maximum(mx_ref[0], m_as)
        mx_ref[1] = jnp.maximum(mx_ref[1], m_ad)

    @pl.when(nb == NBL - 1)
    def _():
        btab_ref[...] = jnp.full((1, 1, 16), _leaky(mx_ref[0] + mx_ref[1]),
                                 jnp.float32)


def _prep1(xp, W1, a_src1, a_dst1):
    return pl.pallas_call(
        _prep1_body,
        grid=(NHEADS, NBL),
        in_specs=[
            pl.BlockSpec((BL, NFEAT), lambda hh, nb: (nb, 0)),
            pl.BlockSpec((1, NFEAT, NHID), lambda hh, nb: (hh, 0, 0)),
            pl.BlockSpec((1, NHID, 1), lambda hh, nb: (hh, 0, 0)),
            pl.BlockSpec((1, NHID, 1), lambda hh, nb: (hh, 0, 0)),
        ],
        out_specs=[
            pl.BlockSpec((BL, 16), lambda hh, nb: (hh * NBL + nb, 0)),
            pl.BlockSpec((1, 1, BL), lambda hh, nb: (hh, 0, nb)),
            pl.BlockSpec((1, 1, BL), lambda hh, nb: (hh, 0, nb)),
            pl.BlockSpec((1, 1, 16), lambda hh, nb: (hh, 0, 0)),
        ],
        out_shape=[
            jax.ShapeDtypeStruct((NHEADS * NP, 16), jnp.float32),
            jax.ShapeDtypeStruct((NHEADS, 1, NP), jnp.float32),
            jax.ShapeDtypeStruct((NHEADS, 1, NP), jnp.float32),
            jax.ShapeDtypeStruct((NHEADS, 1, 16), jnp.float32),
        ],
        scratch_shapes=[pltpu.SMEM((2,), jnp.float32)],
    )(xp, W1, a_src1.reshape(NHEADS, NHID, 1), a_dst1.reshape(NHEADS, NHID, 1))


def _edge_pass_body(NH, NC, C,
                    htab, astab, adtab, btab, src_h, dst_h,
                    acc_o, den_o,
                    as_t, ad_t, bt, acc, den, srcb, dstb, gidxb, wbuf,
                    hrows, sem):
    EPW = E // NC
    NCH = EPW // C
    cid = lax.axis_index("c")
    sid = lax.axis_index("s")
    wid = sid * 2 + cid
    h = wid % NH
    c = wid // NH

    pltpu.sync_copy(astab.at[h, 0], as_t)
    pltpu.sync_copy(adtab.at[h, 0], ad_t)
    pltpu.sync_copy(btab.at[h, 0], bt)

    zf = jnp.zeros((16,), jnp.float32)

    for j in range(NHID):
        @pl.loop(0, NP, step=16, unroll=8)
        def _(i):
            acc[j, pl.ds(i, 16)] = zf

    @pl.loop(0, NP, step=16, unroll=8)
    def _(i):
        den[pl.ds(i, 16)] = zf

    bv = bt[...]
    iot = lax.iota(jnp.int32, 16)
    col8 = lax.rem(iot, 8)
    rep8 = lax.div(iot, 8)
    hoff = h * NP

    @pl.loop(0, NCH)
    def _(k):
        base = c * EPW + k * C
        pltpu.sync_copy(src_h.at[pl.ds(base, C)], srcb)
        pltpu.sync_copy(dst_h.at[pl.ds(base, C)], dstb)

        @pl.loop(0, C, step=16, unroll=4)
        def _(i):
            gidxb[pl.ds(i, 16)] = srcb[pl.ds(i, 16)] + hoff

        pltpu.async_copy(htab.at[gidxb], hrows, sem).wait()

        @pl.loop(0, C, step=16, unroll=2)
        def _(i):
            sv = srcb[pl.ds(i, 16)]
            dv = dstb[pl.ds(i, 16)]
            asv = plsc.load_gather(as_t, [sv])
            adv = plsc.load_gather(ad_t, [dv])
            e = asv + adv
            e = jnp.maximum(e, ALPHA * e) - bv
            w = jnp.exp(e)
            wbuf[pl.ds(i, 16)] = w
            plsc.addupdate_scatter(den, [dv], w)

        @pl.loop(0, C, step=2, unroll=8)
        def _(i):
            iv = jnp.full((16,), i, jnp.int32) + rep8
            hv = plsc.load_gather(hrows, [iv, col8])
            wv = plsc.load_gather(wbuf, [iv])
            dv2 = plsc.load_gather(dstb, [iv])
            prod = hv * wv
            plsc.addupdate_scatter(acc, [col8, dv2], prod)

    pltpu.sync_copy(acc, acc_o.at[c, h])
    pltpu.sync_copy(den, den_o.at[c, h])


def _sc_compiler_params():
    cp = pltpu.CompilerParams()
    fields = pltpu.CompilerParams.__dataclass_fields__
    if "needs_layout_passes" in fields:
        cp = dataclasses.replace(cp, needs_layout_passes=False)
    if "use_tc_tiling_on_sc" in fields:
        cp = dataclasses.replace(cp, use_tc_tiling_on_sc=False)
    return cp


def _make_edge_pass(NH, NC, C):
    mesh = plsc.VectorSubcoreMesh(core_axis_name="c", subcore_axis_name="s")
    return pl.kernel(
        functools.partial(_edge_pass_body, NH, NC, C),
        out_type=[
            jax.ShapeDtypeStruct((NC, NH, NHID, NP), jnp.float32),
            jax.ShapeDtypeStruct((NC, NH, NP), jnp.float32),
        ],
        mesh=mesh,
        scratch_types=[
            pltpu.VMEM((NP,), jnp.float32),
            pltpu.VMEM((NP,), jnp.float32),
            pltpu.VMEM((16,), jnp.float32),
            pltpu.VMEM((NHID, NP), jnp.float32),
            pltpu.VMEM((NP,), jnp.float32),
            pltpu.VMEM((C,), jnp.int32),
            pltpu.VMEM((C,), jnp.int32),
            pltpu.VMEM((C,), jnp.int32),
            pltpu.VMEM((C,), jnp.float32),
            pltpu.VMEM((C, 16), jnp.float32),
            pltpu.SemaphoreType.DMA,
        ],
        compiler_params=_sc_compiler_params(),
    )


def _prep2_body(acc_ref, den_ref, wl1_ref, bl1_ref, w2_ref, a2s_ref, a2d_ref,
                htab_ref, astab_ref, adtab_ref, btab_ref, mx_ref):
    nb = pl.program_id(0)
    a = jnp.sum(acc_ref[...], axis=0)
    d = jnp.sum(den_ref[...], axis=0)
    zs = []
    for h in range(NHEADS):
        oT = a[h] / (d[h][None, :] + 1e-16)
        oT = _elu(oT)
        zT = _elu(_dgt(wl1_ref[h], oT) + bl1_ref[h])
        zs.append(zT)
    hcatT = jnp.concatenate(zs, axis=0)
    h2T = _dgt(w2_ref[...], hcatT)
    h2 = _dgt(h2T, jnp.eye(NHID, dtype=jnp.float32))
    htab_ref[...] = jnp.concatenate(
        [h2, jnp.zeros((BL, NHID), jnp.float32)], axis=1)
    asv = _dgt(a2s_ref[...], h2T)
    adv = _dgt(a2d_ref[...], h2T)
    astab_ref[...] = asv[None]
    adtab_ref[...] = adv[None]
    m_as = jnp.max(asv)
    m_ad = jnp.max(adv)

    @pl.when(nb == 0)
    def _():
        mx_ref[0] = m_as
        mx_ref[1] = m_ad

    @pl.when(nb != 0)
    def _():
        mx_ref[0] = jnp.maximum(mx_ref[0], m_as)
        mx_ref[1] = jnp.maximum(mx_ref[1], m_ad)

    @pl.when(nb == NBL - 1)
    def _():
        btab_ref[...] = jnp.full((1, 1, 16), _leaky(mx_ref[0] + mx_ref[1]),
                                 jnp.float32)


def _prep2(acc1, den1, Wl1, bl1, W2, a2s, a2d):
    return pl.pallas_call(
        _prep2_body,
        grid=(NBL,),
        in_specs=[
            pl.BlockSpec((8, NHEADS, NHID, BL), lambda nb: (0, 0, 0, nb)),
            pl.BlockSpec((8, NHEADS, BL), lambda nb: (0, 0, nb)),
            pl.BlockSpec((NHEADS, NHID, NHID), lambda nb: (0, 0, 0)),
            pl.BlockSpec((NHEADS, NHID, 1), lambda nb: (0, 0, 0)),
            pl.BlockSpec((NHEADS * NHID, NHID), lambda nb: (0, 0)),
            pl.BlockSpec((NHID, 1), lambda nb: (0, 0)),
            pl.BlockSpec((NHID, 1), lambda nb: (0, 0)),
        ],
        out_specs=[
            pl.BlockSpec((BL, 16), lambda nb: (nb, 0)),
            pl.BlockSpec((1, 1, BL), lambda nb: (0, 0, nb)),
            pl.BlockSpec((1, 1, BL), lambda nb: (0, 0, nb)),
            pl.BlockSpec((1, 1, 16), lambda nb: (0, 0, 0)),
        ],
        out_shape=[
            jax.ShapeDtypeStruct((NP, 16), jnp.float32),
            jax.ShapeDtypeStruct((1, 1, NP), jnp.float32),
            jax.ShapeDtypeStruct((1, 1, NP), jnp.float32),
            jax.ShapeDtypeStruct((1, 1, 16), jnp.float32),
        ],
        scratch_shapes=[pltpu.SMEM((2,), jnp.float32)],
    )(acc1, den1, Wl1, bl1.reshape(NHEADS, NHID, 1), W2,
      a2s.reshape(NHID, 1), a2d.reshape(NHID, 1))


def _readout_body(acc_ref, den_ref, wl2_ref, bl2_ref, wlin1_ref, blin1_ref,
                  wlin2_ref, blin2_ref, out_ref, g_ref):
    nb = pl.program_id(0)
    oT = jnp.sum(acc_ref[:, 0], axis=0)
    d = jnp.sum(den_ref[:, 0], axis=0)
    oT = _elu(oT / (d[None, :] + 1e-16))
    t1 = _elu(_dgt(wl2_ref[...], oT) + bl2_ref[...])
    t1 = _elu(t1)
    t2 = _elu(_dgt(wlin1_ref[...], t1) + blin1_ref[...])
    gidx = nb * BL + lax.broadcasted_iota(jnp.int32, (2 * LIN, BL), 1)
    t2 = jnp.where(gidx < N, t2, 0.0)
    gp = jnp.sum(t2, axis=1, keepdims=True)

    @pl.when(nb == 0)
    def _():
        g_ref[...] = gp

    @pl.when(nb != 0)
    def _():
        g_ref[...] = g_ref[...] + gp

    @pl.when(nb == NBL - 1)
    def _():
        logits = (_dgt(g_ref[...], wlin2_ref[...])
                  + blin2_ref[...])
        m = jnp.max(logits)
        ls = logits - m
        out_ref[...] = ls - jnp.log(jnp.sum(jnp.exp(ls)))


def _readout(acc2, den2, Wl2, bl2, W_lin1, b_lin1, W_lin2, b_lin2):
    return pl.pallas_call(
        _readout_body,
        grid=(NBL,),
        in_specs=[
            pl.BlockSpec((NWORKERS, 1, NHID, BL), lambda nb: (0, 0, 0, nb)),
            pl.BlockSpec((NWORKERS, 1, BL), lambda nb: (0, 0, nb)),
            pl.BlockSpec((NHID, LIN), lambda nb: (0, 0)),
            pl.BlockSpec((LIN, 1), lambda nb: (0, 0)),
            pl.BlockSpec((LIN, 2 * LIN), lambda nb: (0, 0)),
            pl.BlockSpec((2 * LIN, 1), lambda nb: (0, 0)),
            pl.BlockSpec((2 * LIN, NCLASS), lambda nb: (0, 0)),
            pl.BlockSpec((1, NCLASS), lambda nb: (0, 0)),
        ],
        out_specs=[pl.BlockSpec((1, NCLASS), lambda nb: (0, 0))],
        out_shape=[jax.ShapeDtypeStruct((1, NCLASS), jnp.float32)],
        scratch_shapes=[pltpu.VMEM((2 * LIN, 1), jnp.float32)],
    )(acc2, den2, Wl2, bl2, W_lin1, b_lin1, W_lin2, b_lin2)


def kernel(x, edge_index, W1, a_src1, a_dst1, Wl1, bl1, W2, a_src2, a_dst2,
           Wl2, bl2, W_lin1, b_lin1, W_lin2, b_lin2):
    src = edge_index[0]
    dst = edge_index[1]
    xp = jnp.pad(x, ((0, NP - N), (0, 0)))

    htab1, astab1, adtab1, btab1 = _prep1(xp, W1, a_src1, a_dst1)

    l1 = _make_edge_pass(NH=NHEADS, NC=8, C=400)
    acc1, den1 = l1(htab1, astab1, adtab1, btab1, src, dst)

    htab2, astab2, adtab2, btab2 = _prep2(acc1, den1, Wl1, bl1, W2,
                                          a_src2, a_dst2)

    l2 = _make_edge_pass(NH=1, NC=NWORKERS, C=400)
    acc2, den2 = l2(htab2, astab2, adtab2, btab2, src, dst)

    (out,) = _readout(acc2, den2,
                      Wl2, bl2.reshape(LIN, 1),
                      W_lin1, b_lin1.reshape(2 * LIN, 1),
                      W_lin2, b_lin2.reshape(1, NCLASS))
    return out


# speedup vs baseline: 51.5359x; 1.2406x over previous
import dataclasses
import functools

import jax
import jax.numpy as jnp
from jax import lax
from jax.experimental import pallas as pl
from jax.experimental.pallas import tpu as pltpu
from jax.experimental.pallas import tpu_sc as plsc

N = 10000
NP = 10240
E = 320000
NFEAT = 128
NHID = 8
NHEADS = 4
LIN = 64
NCLASS = 10
ALPHA = 0.2

BL = 1024
NBL = NP // BL
NWORKERS = 32


def _elu(v):
    return jnp.where(v > 0, v, jnp.exp(jnp.minimum(v, 0.0)) - 1.0)


def _leaky(v):
    return jnp.maximum(v, ALPHA * v)


def _dgt(a, b):
    return lax.dot_general(a, b, (((0,), (0,)), ((), ())),
                           preferred_element_type=jnp.float32)


def _prep1_body(x_ref, w1_ref, as1_ref, ad1_ref,
                htab_ref, astab_ref, adtab_ref, btab_ref, mx_ref):
    nb = pl.program_id(1)
    hh = pl.program_id(0)
    xb = x_ref[...]
    hb = jnp.dot(xb, w1_ref[0], preferred_element_type=jnp.float32)
    htab_ref[...] = jnp.concatenate(
        [hb, jnp.zeros((BL, NHID), jnp.float32)], axis=1)
    va_s = jnp.dot(w1_ref[0], as1_ref[0],
                   preferred_element_type=jnp.float32)
    va_d = jnp.dot(w1_ref[0], ad1_ref[0],
                   preferred_element_type=jnp.float32)
    asv = lax.dot_general(va_s, xb, (((0,), (1,)), ((), ())),
                          preferred_element_type=jnp.float32)
    adv = lax.dot_general(va_d, xb, (((0,), (1,)), ((), ())),
                          preferred_element_type=jnp.float32)
    astab_ref[...] = asv[None]
    adtab_ref[...] = adv[None]
    m_as = jnp.max(asv)
    m_ad = jnp.max(adv)

    @pl.when(nb == 0)
    def _():
        mx_ref[0] = m_as
        mx_ref[1] = m_ad

    @pl.when(nb != 0)
    def _():
        mx_ref[0] = jnp.maximum(mx_ref[0], m_as)
        mx_ref[1] = jnp.maximum(mx_ref[1], m_ad)

    @pl.when(nb == NBL - 1)
    def _():
        btab_ref[...] = jnp.full((1, 1, 16), _leaky(mx_ref[0] + mx_ref[1]),
                                 jnp.float32)


def _prep1(xp, W1, a_src1, a_dst1):
    return pl.pallas_call(
        _prep1_body,
        grid=(NHEADS, NBL),
        in_specs=[
            pl.BlockSpec((BL, NFEAT), lambda hh, nb: (nb, 0)),
            pl.BlockSpec((1, NFEAT, NHID), lambda hh, nb: (hh, 0, 0)),
            pl.BlockSpec((1, NHID, 1), lambda hh, nb: (hh, 0, 0)),
            pl.BlockSpec((1, NHID, 1), lambda hh, nb: (hh, 0, 0)),
        ],
        out_specs=[
            pl.BlockSpec((BL, 16), lambda hh, nb: (hh * NBL + nb, 0)),
            pl.BlockSpec((1, 1, BL), lambda hh, nb: (hh, 0, nb)),
            pl.BlockSpec((1, 1, BL), lambda hh, nb: (hh, 0, nb)),
            pl.BlockSpec((1, 1, 16), lambda hh, nb: (hh, 0, 0)),
        ],
        out_shape=[
            jax.ShapeDtypeStruct((NHEADS * NP, 16), jnp.float32),
            jax.ShapeDtypeStruct((NHEADS, 1, NP), jnp.float32),
            jax.ShapeDtypeStruct((NHEADS, 1, NP), jnp.float32),
            jax.ShapeDtypeStruct((NHEADS, 1, 16), jnp.float32),
        ],
        scratch_shapes=[pltpu.SMEM((2,), jnp.float32)],
    )(xp, W1, a_src1.reshape(NHEADS, NHID, 1), a_dst1.reshape(NHEADS, NHID, 1))


def _edge_pass_body(NH, NC, C,
                    htab, astab, adtab, btab, src_h, dst_h,
                    acc_o, den_o,
                    as_t, ad_t, bt, acc, den, srcb, dstb, gidxb, wbuf,
                    hrows, sem):
    EPW = E // NC
    NCH = EPW // C
    cid = lax.axis_index("c")
    sid = lax.axis_index("s")
    wid = sid * 2 + cid
    h = wid % NH
    c = wid // NH

    pltpu.sync_copy(astab.at[h, 0], as_t)
    pltpu.sync_copy(adtab.at[h, 0], ad_t)
    pltpu.sync_copy(btab.at[h, 0], bt)

    zf = jnp.zeros((16,), jnp.float32)

    for j in range(NHID):
        @plsc.parallel_loop(0, NP, 16, unroll=8)
        def _(i):
            acc[j, pl.ds(i, 16)] = zf

    @plsc.parallel_loop(0, NP, 16, unroll=8)
    def _(i):
        den[pl.ds(i, 16)] = zf

    bv = bt[...]
    iot = lax.iota(jnp.int32, 16)
    col8 = lax.rem(iot, 8)
    rep8 = lax.div(iot, 8)
    hoff = h * NP

    @pl.loop(0, NCH)
    def _(k):
        base = c * EPW + k * C
        pltpu.sync_copy(src_h.at[pl.ds(base, C)], srcb)
        pltpu.sync_copy(dst_h.at[pl.ds(base, C)], dstb)

        @plsc.parallel_loop(0, C, 16, unroll=4)
        def _(i):
            gidxb[pl.ds(i, 16)] = srcb[pl.ds(i, 16)] + hoff

        pltpu.async_copy(htab.at[gidxb], hrows, sem).wait()

        @plsc.parallel_loop(0, C, 16, unroll=2)
        def _(i):
            sv = srcb[pl.ds(i, 16)]
            dv = dstb[pl.ds(i, 16)]
            asv = plsc.load_gather(as_t, [sv])
            adv = plsc.load_gather(ad_t, [dv])
            e = asv + adv
            e = jnp.maximum(e, ALPHA * e) - bv
            w = jnp.exp(e)
            wbuf[pl.ds(i, 16)] = w
            plsc.addupdate_scatter(den, [dv], w)

        @plsc.parallel_loop(0, C, 2, unroll=8, carry=rep8)
        def _(i, iv):
            hv = plsc.load_gather(hrows, [iv, col8])
            wv = plsc.load_gather(wbuf, [iv])
            dv2 = plsc.load_gather(dstb, [iv])
            prod = hv * wv
            plsc.addupdate_scatter(acc, [col8, dv2], prod)
            return iv + 2

    pltpu.sync_copy(acc, acc_o.at[c, h])
    pltpu.sync_copy(den, den_o.at[c, h])


def _sc_compiler_params():
    cp = pltpu.CompilerParams()
    fields = pltpu.CompilerParams.__dataclass_fields__
    if "needs_layout_passes" in fields:
        cp = dataclasses.replace(cp, needs_layout_passes=False)
    if "use_tc_tiling_on_sc" in fields:
        cp = dataclasses.replace(cp, use_tc_tiling_on_sc=False)
    return cp


def _make_edge_pass(NH, NC, C):
    mesh = plsc.VectorSubcoreMesh(core_axis_name="c", subcore_axis_name="s")
    return pl.kernel(
        functools.partial(_edge_pass_body, NH, NC, C),
        out_type=[
            jax.ShapeDtypeStruct((NC, NH, NHID, NP), jnp.float32),
            jax.ShapeDtypeStruct((NC, NH, NP), jnp.float32),
        ],
        mesh=mesh,
        scratch_types=[
            pltpu.VMEM((NP,), jnp.float32),
            pltpu.VMEM((NP,), jnp.float32),
            pltpu.VMEM((16,), jnp.float32),
            pltpu.VMEM((NHID, NP), jnp.float32),
            pltpu.VMEM((NP,), jnp.float32),
            pltpu.VMEM((C,), jnp.int32),
            pltpu.VMEM((C,), jnp.int32),
            pltpu.VMEM((C,), jnp.int32),
            pltpu.VMEM((C,), jnp.float32),
            pltpu.VMEM((C, 16), jnp.float32),
            pltpu.SemaphoreType.DMA,
        ],
        compiler_params=_sc_compiler_params(),
    )


def _prep2_body(acc_ref, den_ref, wl1_ref, bl1_ref, w2_ref, a2s_ref, a2d_ref,
                htab_ref, astab_ref, adtab_ref, btab_ref, mx_ref):
    nb = pl.program_id(0)
    a = jnp.sum(acc_ref[...], axis=0)
    d = jnp.sum(den_ref[...], axis=0)
    zs = []
    for h in range(NHEADS):
        oT = a[h] / (d[h][None, :] + 1e-16)
        oT = _elu(oT)
        zT = _elu(_dgt(wl1_ref[h], oT) + bl1_ref[h])
        zs.append(zT)
    hcatT = jnp.concatenate(zs, axis=0)
    h2T = _dgt(w2_ref[...], hcatT)
    h2 = _dgt(h2T, jnp.eye(NHID, dtype=jnp.float32))
    htab_ref[...] = jnp.concatenate(
        [h2, jnp.zeros((BL, NHID), jnp.float32)], axis=1)
    asv = _dgt(a2s_ref[...], h2T)
    adv = _dgt(a2d_ref[...], h2T)
    astab_ref[...] = asv[None]
    adtab_ref[...] = adv[None]
    m_as = jnp.max(asv)
    m_ad = jnp.max(adv)

    @pl.when(nb == 0)
    def _():
        mx_ref[0] = m_as
        mx_ref[1] = m_ad

    @pl.when(nb != 0)
    def _():
        mx_ref[0] = jnp.maximum(mx_ref[0], m_as)
        mx_ref[1] = jnp.maximum(mx_ref[1], m_ad)

    @pl.when(nb == NBL - 1)
    def _():
        btab_ref[...] = jnp.full((1, 1, 16), _leaky(mx_ref[0] + mx_ref[1]),
                                 jnp.float32)


def _prep2(acc1, den1, Wl1, bl1, W2, a2s, a2d):
    return pl.pallas_call(
        _prep2_body,
        grid=(NBL,),
        in_specs=[
            pl.BlockSpec((8, NHEADS, NHID, BL), lambda nb: (0, 0, 0, nb)),
            pl.BlockSpec((8, NHEADS, BL), lambda nb: (0, 0, nb)),
            pl.BlockSpec((NHEADS, NHID, NHID), lambda nb: (0, 0, 0)),
            pl.BlockSpec((NHEADS, NHID, 1), lambda nb: (0, 0, 0)),
            pl.BlockSpec((NHEADS * NHID, NHID), lambda nb: (0, 0)),
            pl.BlockSpec((NHID, 1), lambda nb: (0, 0)),
            pl.BlockSpec((NHID, 1), lambda nb: (0, 0)),
        ],
        out_specs=[
            pl.BlockSpec((BL, 16), lambda nb: (nb, 0)),
            pl.BlockSpec((1, 1, BL), lambda nb: (0, 0, nb)),
            pl.BlockSpec((1, 1, BL), lambda nb: (0, 0, nb)),
            pl.BlockSpec((1, 1, 16), lambda nb: (0, 0, 0)),
        ],
        out_shape=[
            jax.ShapeDtypeStruct((NP, 16), jnp.float32),
            jax.ShapeDtypeStruct((1, 1, NP), jnp.float32),
            jax.ShapeDtypeStruct((1, 1, NP), jnp.float32),
            jax.ShapeDtypeStruct((1, 1, 16), jnp.float32),
        ],
        scratch_shapes=[pltpu.SMEM((2,), jnp.float32)],
    )(acc1, den1, Wl1, bl1.reshape(NHEADS, NHID, 1), W2,
      a2s.reshape(NHID, 1), a2d.reshape(NHID, 1))


def _readout_body(acc_ref, den_ref, wl2_ref, bl2_ref, wlin1_ref, blin1_ref,
                  wlin2_ref, blin2_ref, out_ref, g_ref):
    nb = pl.program_id(0)
    oT = jnp.sum(acc_ref[:, 0], axis=0)
    d = jnp.sum(den_ref[:, 0], axis=0)
    oT = _elu(oT / (d[None, :] + 1e-16))
    t1 = _elu(_dgt(wl2_ref[...], oT) + bl2_ref[...])
    t1 = _elu(t1)
    t2 = _elu(_dgt(wlin1_ref[...], t1) + blin1_ref[...])
    gidx = nb * BL + lax.broadcasted_iota(jnp.int32, (2 * LIN, BL), 1)
    t2 = jnp.where(gidx < N, t2, 0.0)
    gp = jnp.sum(t2, axis=1, keepdims=True)

    @pl.when(nb == 0)
    def _():
        g_ref[...] = gp

    @pl.when(nb != 0)
    def _():
        g_ref[...] = g_ref[...] + gp

    @pl.when(nb == NBL - 1)
    def _():
        logits = (_dgt(g_ref[...], wlin2_ref[...])
                  + blin2_ref[...])
        m = jnp.max(logits)
        ls = logits - m
        out_ref[...] = ls - jnp.log(jnp.sum(jnp.exp(ls)))


def _readout(acc2, den2, Wl2, bl2, W_lin1, b_lin1, W_lin2, b_lin2):
    return pl.pallas_call(
        _readout_body,
        grid=(NBL,),
        in_specs=[
            pl.BlockSpec((NWORKERS, 1, NHID, BL), lambda nb: (0, 0, 0, nb)),
            pl.BlockSpec((NWORKERS, 1, BL), lambda nb: (0, 0, nb)),
            pl.BlockSpec((NHID, LIN), lambda nb: (0, 0)),
            pl.BlockSpec((LIN, 1), lambda nb: (0, 0)),
            pl.BlockSpec((LIN, 2 * LIN), lambda nb: (0, 0)),
            pl.BlockSpec((2 * LIN, 1), lambda nb: (0, 0)),
            pl.BlockSpec((2 * LIN, NCLASS), lambda nb: (0, 0)),
            pl.BlockSpec((1, NCLASS), lambda nb: (0, 0)),
        ],
        out_specs=[pl.BlockSpec((1, NCLASS), lambda nb: (0, 0))],
        out_shape=[jax.ShapeDtypeStruct((1, NCLASS), jnp.float32)],
        scratch_shapes=[pltpu.VMEM((2 * LIN, 1), jnp.float32)],
    )(acc2, den2, Wl2, bl2, W_lin1, b_lin1, W_lin2, b_lin2)


def kernel(x, edge_index, W1, a_src1, a_dst1, Wl1, bl1, W2, a_src2, a_dst2,
           Wl2, bl2, W_lin1, b_lin1, W_lin2, b_lin2):
    src = edge_index[0]
    dst = edge_index[1]
    xp = jnp.pad(x, ((0, NP - N), (0, 0)))

    htab1, astab1, adtab1, btab1 = _prep1(xp, W1, a_src1, a_dst1)

    l1 = _make_edge_pass(NH=NHEADS, NC=8, C=400)
    acc1, den1 = l1(htab1, astab1, adtab1, btab1, src, dst)

    htab2, astab2, adtab2, btab2 = _prep2(acc1, den1, Wl1, bl1, W2,
                                          a_src2, a_dst2)

    l2 = _make_edge_pass(NH=1, NC=NWORKERS, C=400)
    acc2, den2 = l2(htab2, astab2, adtab2, btab2, src, dst)

    (out,) = _readout(acc2, den2,
                      Wl2, bl2.reshape(LIN, 1),
                      W_lin1, b_lin1.reshape(2 * LIN, 1),
                      W_lin2, b_lin2.reshape(1, NCLASS))
    return out


# speedup vs baseline: 61.9812x; 1.2027x over previous
import dataclasses
import functools

import jax
import jax.numpy as jnp
from jax import lax
from jax.experimental import pallas as pl
from jax.experimental.pallas import tpu as pltpu
from jax.experimental.pallas import tpu_sc as plsc

N = 10000
NP = 10240
E = 320000
NFEAT = 128
NHID = 8
NHEADS = 4
LIN = 64
NCLASS = 10
ALPHA = 0.2

BL = 1024
NBL = NP // BL
NWORKERS = 32


def _elu(v):
    return jnp.where(v > 0, v, jnp.exp(jnp.minimum(v, 0.0)) - 1.0)


def _leaky(v):
    return jnp.maximum(v, ALPHA * v)


def _dgt(a, b):
    return lax.dot_general(a, b, (((0,), (0,)), ((), ())),
                           preferred_element_type=jnp.float32)


def _prep1_body(x_ref, w1_ref, as1_ref, ad1_ref,
                htab_ref, astab_ref, adtab_ref, btab_ref, mx_ref):
    nb = pl.program_id(1)
    hh = pl.program_id(0)
    xb = x_ref[...]
    hb = jnp.dot(xb, w1_ref[0], preferred_element_type=jnp.float32)
    htab_ref[...] = jnp.concatenate(
        [hb, jnp.zeros((BL, NHID), jnp.float32)], axis=1)
    va_s = jnp.dot(w1_ref[0], as1_ref[0],
                   preferred_element_type=jnp.float32)
    va_d = jnp.dot(w1_ref[0], ad1_ref[0],
                   preferred_element_type=jnp.float32)
    asv = lax.dot_general(va_s, xb, (((0,), (1,)), ((), ())),
                          preferred_element_type=jnp.float32)
    adv = lax.dot_general(va_d, xb, (((0,), (1,)), ((), ())),
                          preferred_element_type=jnp.float32)
    astab_ref[...] = asv[None]
    adtab_ref[...] = adv[None]
    m_as = jnp.max(asv)
    m_ad = jnp.max(adv)

    @pl.when(nb == 0)
    def _():
        mx_ref[0] = m_as
        mx_ref[1] = m_ad

    @pl.when(nb != 0)
    def _():
        mx_ref[0] = jnp.maximum(mx_ref[0], m_as)
        mx_ref[1] = jnp.maximum(mx_ref[1], m_ad)

    @pl.when(nb == NBL - 1)
    def _():
        btab_ref[...] = jnp.full((1, 1, 16), _leaky(mx_ref[0] + mx_ref[1]),
                                 jnp.float32)


def _prep1(xp, W1, a_src1, a_dst1):
    return pl.pallas_call(
        _prep1_body,
        grid=(NHEADS, NBL),
        in_specs=[
            pl.BlockSpec((BL, NFEAT), lambda hh, nb: (nb, 0)),
            pl.BlockSpec((1, NFEAT, NHID), lambda hh, nb: (hh, 0, 0)),
            pl.BlockSpec((1, NHID, 1), lambda hh, nb: (hh, 0, 0)),
            pl.BlockSpec((1, NHID, 1), lambda hh, nb: (hh, 0, 0)),
        ],
        out_specs=[
            pl.BlockSpec((BL, 16), lambda hh, nb: (hh * NBL + nb, 0)),
            pl.BlockSpec((1, 1, BL), lambda hh, nb: (hh, 0, nb)),
            pl.BlockSpec((1, 1, BL), lambda hh, nb: (hh, 0, nb)),
            pl.BlockSpec((1, 1, 16), lambda hh, nb: (hh, 0, 0)),
        ],
        out_shape=[
            jax.ShapeDtypeStruct((NHEADS * NP, 16), jnp.float32),
            jax.ShapeDtypeStruct((NHEADS, 1, NP), jnp.float32),
            jax.ShapeDtypeStruct((NHEADS, 1, NP), jnp.float32),
            jax.ShapeDtypeStruct((NHEADS, 1, 16), jnp.float32),
        ],
        scratch_shapes=[pltpu.SMEM((2,), jnp.float32)],
    )(xp, W1, a_src1.reshape(NHEADS, NHID, 1), a_dst1.reshape(NHEADS, NHID, 1))


C = 800


def _edge_pass_body(NH, NC,
                    htab, astab, adtab, btab, src_h, dst_h,
                    acc_o, den_o,
                    as_t, ad_t, bt, acc, den, srcb, dstb, gidxb, wbuf,
                    hrows, sem0, sem1):
    EPW = E // NC
    NCH = EPW // C
    cid = lax.axis_index("c")
    sid = lax.axis_index("s")
    wid = sid * 2 + cid
    h = wid % NH
    c = wid // NH

    pltpu.sync_copy(astab.at[h, 0], as_t)
    pltpu.sync_copy(adtab.at[h, 0], ad_t)
    pltpu.sync_copy(btab.at[h, 0], bt)

    zf = jnp.zeros((16,), jnp.float32)

    for j in range(NHID):
        @plsc.parallel_loop(0, NP, 16, unroll=8)
        def _(i):
            acc[j, pl.ds(i, 16)] = zf

    @plsc.parallel_loop(0, NP, 16, unroll=8)
    def _(i):
        den[pl.ds(i, 16)] = zf

    bv = bt[...]
    iot = lax.iota(jnp.int32, 16)
    col8 = lax.rem(iot, 8)
    rep8 = lax.div(iot, 8)
    hoff = h * NP

    @pl.loop(0, NCH)
    def _(k):
        base = c * EPW + k * C
        pltpu.sync_copy(src_h.at[pl.ds(base, C)], srcb)
        pltpu.sync_copy(dst_h.at[pl.ds(base, C)], dstb)

        @plsc.parallel_loop(0, C, 16, unroll=4)
        def _(i):
            gidxb[pl.ds(i, 16)] = srcb[pl.ds(i, 16)] + hoff

        pltpu.async_copy(htab.at[gidxb], hrows, sem0).wait()

        @plsc.parallel_loop(0, C, 16, unroll=2)
        def _(i):
            sv = srcb[pl.ds(i, 16)]
            dv = dstb[pl.ds(i, 16)]
            asv = plsc.load_gather(as_t, [sv])
            adv = plsc.load_gather(ad_t, [dv])
            e = asv + adv
            e = jnp.maximum(e, ALPHA * e) - bv
            w = jnp.exp(e)
            wbuf[pl.ds(i, 16)] = w
            plsc.addupdate_scatter(den, [dv], w)

        @plsc.parallel_loop(0, C, 2, unroll=8, carry=rep8)
        def _(i, iv):
            hv = plsc.load_gather(hrows, [iv, col8])
            wv = plsc.load_gather(wbuf, [iv])
            dv2 = plsc.load_gather(dstb, [iv])
            prod = hv * wv
            plsc.addupdate_scatter(acc, [col8, dv2], prod)
            return iv + 2

    pltpu.sync_copy(acc, acc_o.at[c, h])
    pltpu.sync_copy(den, den_o.at[c, h])


def _sc_compiler_params():
    cp = pltpu.CompilerParams()
    fields = pltpu.CompilerParams.__dataclass_fields__
    if "needs_layout_passes" in fields:
        cp = dataclasses.replace(cp, needs_layout_passes=False)
    if "use_tc_tiling_on_sc" in fields:
        cp = dataclasses.replace(cp, use_tc_tiling_on_sc=False)
    return cp


def _make_edge_pass(NH, NC):
    mesh = plsc.VectorSubcoreMesh(core_axis_name="c", subcore_axis_name="s")
    return pl.kernel(
        functools.partial(_edge_pass_body, NH, NC),
        out_type=[
            jax.ShapeDtypeStruct((NC, NH, NHID, NP), jnp.float32),
            jax.ShapeDtypeStruct((NC, NH, NP), jnp.float32),
        ],
        mesh=mesh,
        scratch_types=[
            pltpu.VMEM((NP,), jnp.float32),
            pltpu.VMEM((NP,), jnp.float32),
            pltpu.VMEM((16,), jnp.float32),
            pltpu.VMEM((NHID, NP), jnp.float32),
            pltpu.VMEM((NP,), jnp.float32),
            pltpu.VMEM((C,), jnp.int32),
            pltpu.VMEM((C,), jnp.int32),
            pltpu.VMEM((C,), jnp.int32),
            pltpu.VMEM((C,), jnp.float32),
            pltpu.VMEM((C, 16), jnp.float32),
            pltpu.SemaphoreType.DMA,
            pltpu.SemaphoreType.DMA,
        ],
        compiler_params=_sc_compiler_params(),
    )


def _prep2_body(acc_ref, den_ref, wl1_ref, bl1_ref, w2_ref, a2s_ref, a2d_ref,
                htab_ref, astab_ref, adtab_ref, btab_ref, mx_ref):
    nb = pl.program_id(0)
    a = jnp.sum(acc_ref[...], axis=0)
    d = jnp.sum(den_ref[...], axis=0)
    zs = []
    for h in range(NHEADS):
        oT = a[h] / (d[h][None, :] + 1e-16)
        oT = _elu(oT)
        zT = _elu(_dgt(wl1_ref[h], oT) + bl1_ref[h])
        zs.append(zT)
    hcatT = jnp.concatenate(zs, axis=0)
    h2T = _dgt(w2_ref[...], hcatT)
    h2 = _dgt(h2T, jnp.eye(NHID, dtype=jnp.float32))
    htab_ref[...] = jnp.concatenate(
        [h2, jnp.zeros((BL, NHID), jnp.float32)], axis=1)
    asv = _dgt(a2s_ref[...], h2T)
    adv = _dgt(a2d_ref[...], h2T)
    astab_ref[...] = asv[None]
    adtab_ref[...] = adv[None]
    m_as = jnp.max(asv)
    m_ad = jnp.max(adv)

    @pl.when(nb == 0)
    def _():
        mx_ref[0] = m_as
        mx_ref[1] = m_ad

    @pl.when(nb != 0)
    def _():
        mx_ref[0] = jnp.maximum(mx_ref[0], m_as)
        mx_ref[1] = jnp.maximum(mx_ref[1], m_ad)

    @pl.when(nb == NBL - 1)
    def _():
        btab_ref[...] = jnp.full((1, 1, 16), _leaky(mx_ref[0] + mx_ref[1]),
                                 jnp.float32)


def _prep2(acc1, den1, Wl1, bl1, W2, a2s, a2d):
    return pl.pallas_call(
        _prep2_body,
        grid=(NBL,),
        in_specs=[
            pl.BlockSpec((8, NHEADS, NHID, BL), lambda nb: (0, 0, 0, nb)),
            pl.BlockSpec((8, NHEADS, BL), lambda nb: (0, 0, nb)),
            pl.BlockSpec((NHEADS, NHID, NHID), lambda nb: (0, 0, 0)),
            pl.BlockSpec((NHEADS, NHID, 1), lambda nb: (0, 0, 0)),
            pl.BlockSpec((NHEADS * NHID, NHID), lambda nb: (0, 0)),
            pl.BlockSpec((NHID, 1), lambda nb: (0, 0)),
            pl.BlockSpec((NHID, 1), lambda nb: (0, 0)),
        ],
        out_specs=[
            pl.BlockSpec((BL, 16), lambda nb: (nb, 0)),
            pl.BlockSpec((1, 1, BL), lambda nb: (0, 0, nb)),
            pl.BlockSpec((1, 1, BL), lambda nb: (0, 0, nb)),
            pl.BlockSpec((1, 1, 16), lambda nb: (0, 0, 0)),
        ],
        out_shape=[
            jax.ShapeDtypeStruct((NP, 16), jnp.float32),
            jax.ShapeDtypeStruct((1, 1, NP), jnp.float32),
            jax.ShapeDtypeStruct((1, 1, NP), jnp.float32),
            jax.ShapeDtypeStruct((1, 1, 16), jnp.float32),
        ],
        scratch_shapes=[pltpu.SMEM((2,), jnp.float32)],
    )(acc1, den1, Wl1, bl1.reshape(NHEADS, NHID, 1), W2,
      a2s.reshape(NHID, 1), a2d.reshape(NHID, 1))


def _readout_body(acc_ref, den_ref, wl2_ref, bl2_ref, wlin1_ref, blin1_ref,
                  wlin2_ref, blin2_ref, out_ref, g_ref):
    nb = pl.program_id(0)
    oT = jnp.sum(acc_ref[:, 0], axis=0)
    d = jnp.sum(den_ref[:, 0], axis=0)
    oT = _elu(oT / (d[None, :] + 1e-16))
    t1 = _elu(_dgt(wl2_ref[...], oT) + bl2_ref[...])
    t1 = _elu(t1)
    t2 = _elu(_dgt(wlin1_ref[...], t1) + blin1_ref[...])
    gidx = nb * BL + lax.broadcasted_iota(jnp.int32, (2 * LIN, BL), 1)
    t2 = jnp.where(gidx < N, t2, 0.0)
    gp = jnp.sum(t2, axis=1, keepdims=True)

    @pl.when(nb == 0)
    def _():
        g_ref[...] = gp

    @pl.when(nb != 0)
    def _():
        g_ref[...] = g_ref[...] + gp

    @pl.when(nb == NBL - 1)
    def _():
        logits = (_dgt(g_ref[...], wlin2_ref[...])
                  + blin2_ref[...])
        m = jnp.max(logits)
        ls = logits - m
        out_ref[...] = ls - jnp.log(jnp.sum(jnp.exp(ls)))


def _readout(acc2, den2, Wl2, bl2, W_lin1, b_lin1, W_lin2, b_lin2):
    return pl.pallas_call(
        _readout_body,
        grid=(NBL,),
        in_specs=[
            pl.BlockSpec((NWORKERS, 1, NHID, BL), lambda nb: (0, 0, 0, nb)),
            pl.BlockSpec((NWORKERS, 1, BL), lambda nb: (0, 0, nb)),
            pl.BlockSpec((NHID, LIN), lambda nb: (0, 0)),
            pl.BlockSpec((LIN, 1), lambda nb: (0, 0)),
            pl.BlockSpec((LIN, 2 * LIN), lambda nb: (0, 0)),
            pl.BlockSpec((2 * LIN, 1), lambda nb: (0, 0)),
            pl.BlockSpec((2 * LIN, NCLASS), lambda nb: (0, 0)),
            pl.BlockSpec((1, NCLASS), lambda nb: (0, 0)),
        ],
        out_specs=[pl.BlockSpec((1, NCLASS), lambda nb: (0, 0))],
        out_shape=[jax.ShapeDtypeStruct((1, NCLASS), jnp.float32)],
        scratch_shapes=[pltpu.VMEM((2 * LIN, 1), jnp.float32)],
    )(acc2, den2, Wl2, bl2, W_lin1, b_lin1, W_lin2, b_lin2)


def kernel(x, edge_index, W1, a_src1, a_dst1, Wl1, bl1, W2, a_src2, a_dst2,
           Wl2, bl2, W_lin1, b_lin1, W_lin2, b_lin2):
    src = edge_index[0]
    dst = edge_index[1]
    xp = jnp.pad(x, ((0, NP - N), (0, 0)))

    htab1, astab1, adtab1, btab1 = _prep1(xp, W1, a_src1, a_dst1)

    l1 = _make_edge_pass(NH=NHEADS, NC=8)
    acc1, den1 = l1(htab1, astab1, adtab1, btab1, src, dst)

    htab2, astab2, adtab2, btab2 = _prep2(acc1, den1, Wl1, bl1, W2,
                                          a_src2, a_dst2)

    l2 = _make_edge_pass(NH=1, NC=NWORKERS)
    acc2, den2 = l2(htab2, astab2, adtab2, btab2, src, dst)

    (out,) = _readout(acc2, den2,
                      Wl2, bl2.reshape(LIN, 1),
                      W_lin1, b_lin1.reshape(2 * LIN, 1),
                      W_lin2, b_lin2.reshape(1, NCLASS))
    return out


# speedup vs baseline: 68.2627x; 1.1013x over previous
import dataclasses
import functools

import jax
import jax.numpy as jnp
from jax import lax
from jax.experimental import pallas as pl
from jax.experimental.pallas import tpu as pltpu
from jax.experimental.pallas import tpu_sc as plsc

N = 10000
NP = 10240
E = 320000
NFEAT = 128
NHID = 8
NHEADS = 4
LIN = 64
NCLASS = 10
ALPHA = 0.2

BL = 1024
NBL = NP // BL
NWORKERS = 32


def _elu(v):
    return jnp.where(v > 0, v, jnp.exp(jnp.minimum(v, 0.0)) - 1.0)


def _leaky(v):
    return jnp.maximum(v, ALPHA * v)


def _dgt(a, b):
    return lax.dot_general(a, b, (((0,), (0,)), ((), ())),
                           preferred_element_type=jnp.float32)


def _prep1_body(x_ref, w1_ref, as1_ref, ad1_ref,
                htab_ref, astab_ref, adtab_ref, btab_ref, mx_ref):
    nb = pl.program_id(1)
    hh = pl.program_id(0)
    xb = x_ref[...]
    hb = jnp.dot(xb, w1_ref[0], preferred_element_type=jnp.float32)
    htab_ref[...] = jnp.concatenate(
        [hb, jnp.zeros((BL, NHID), jnp.float32)], axis=1)
    va_s = jnp.dot(w1_ref[0], as1_ref[0],
                   preferred_element_type=jnp.float32)
    va_d = jnp.dot(w1_ref[0], ad1_ref[0],
                   preferred_element_type=jnp.float32)
    asv = lax.dot_general(va_s, xb, (((0,), (1,)), ((), ())),
                          preferred_element_type=jnp.float32)
    adv = lax.dot_general(va_d, xb, (((0,), (1,)), ((), ())),
                          preferred_element_type=jnp.float32)
    astab_ref[...] = asv[None]
    adtab_ref[...] = adv[None]
    m_as = jnp.max(asv)
    m_ad = jnp.max(adv)

    @pl.when(nb == 0)
    def _():
        mx_ref[0] = m_as
        mx_ref[1] = m_ad

    @pl.when(nb != 0)
    def _():
        mx_ref[0] = jnp.maximum(mx_ref[0], m_as)
        mx_ref[1] = jnp.maximum(mx_ref[1], m_ad)

    @pl.when(nb == NBL - 1)
    def _():
        btab_ref[...] = jnp.full((1, 1, 16), _leaky(mx_ref[0] + mx_ref[1]),
                                 jnp.float32)


def _prep1(xp, W1, a_src1, a_dst1):
    return pl.pallas_call(
        _prep1_body,
        grid=(NHEADS, NBL),
        in_specs=[
            pl.BlockSpec((BL, NFEAT), lambda hh, nb: (nb, 0)),
            pl.BlockSpec((1, NFEAT, NHID), lambda hh, nb: (hh, 0, 0)),
            pl.BlockSpec((1, NHID, 1), lambda hh, nb: (hh, 0, 0)),
            pl.BlockSpec((1, NHID, 1), lambda hh, nb: (hh, 0, 0)),
        ],
        out_specs=[
            pl.BlockSpec((BL, 16), lambda hh, nb: (hh * NBL + nb, 0)),
            pl.BlockSpec((1, 1, BL), lambda hh, nb: (hh, 0, nb)),
            pl.BlockSpec((1, 1, BL), lambda hh, nb: (hh, 0, nb)),
            pl.BlockSpec((1, 1, 16), lambda hh, nb: (hh, 0, 0)),
        ],
        out_shape=[
            jax.ShapeDtypeStruct((NHEADS * NP, 16), jnp.float32),
            jax.ShapeDtypeStruct((NHEADS, 1, NP), jnp.float32),
            jax.ShapeDtypeStruct((NHEADS, 1, NP), jnp.float32),
            jax.ShapeDtypeStruct((NHEADS, 1, 16), jnp.float32),
        ],
        scratch_shapes=[pltpu.SMEM((2,), jnp.float32)],
    )(xp, W1, a_src1.reshape(NHEADS, NHID, 1), a_dst1.reshape(NHEADS, NHID, 1))


C = 800


def _edge_pass_body(NH, NC,
                    htab, astab, adtab, btab, src_h, dst_h,
                    acc_o, den_o,
                    as_t, ad_t, bt, acc, den, srcb, dstb, gidxb, wbuf,
                    hrows, sem0, sem1):
    EPW = E // NC
    NCH = EPW // C
    cid = lax.axis_index("c")
    sid = lax.axis_index("s")
    wid = sid * 2 + cid
    h = wid % NH
    c = wid // NH

    pltpu.sync_copy(astab.at[h, 0], as_t)
    pltpu.sync_copy(adtab.at[h, 0], ad_t)
    pltpu.sync_copy(btab.at[h, 0], bt)

    zf = jnp.zeros((16,), jnp.float32)

    for j in range(NHID):
        @plsc.parallel_loop(0, NP, 16, unroll=8)
        def _(i):
            acc[j, pl.ds(i, 16)] = zf

    @plsc.parallel_loop(0, NP, 16, unroll=8)
    def _(i):
        den[pl.ds(i, 16)] = zf

    bv = bt[...]
    iot = lax.iota(jnp.int32, 16)
    col8 = lax.rem(iot, 8)
    rep8 = lax.div(iot, 8)
    hoff = h * NP

    @pl.loop(0, NCH)
    def _(k):
        base = c * EPW + k * C
        dsrc = pltpu.async_copy(src_h.at[pl.ds(base, C)], srcb, sem0)
        ddst = pltpu.async_copy(dst_h.at[pl.ds(base, C)], dstb, sem1)
        dsrc.wait()

        @plsc.parallel_loop(0, C, 16, unroll=4)
        def _(i):
            gidxb[pl.ds(i, 16)] = srcb[pl.ds(i, 16)] + hoff

        dgat = pltpu.async_copy(htab.at[gidxb], hrows, sem0)
        ddst.wait()

        @plsc.parallel_loop(0, C, 16, unroll=2)
        def _(i):
            sv = srcb[pl.ds(i, 16)]
            dv = dstb[pl.ds(i, 16)]
            asv = plsc.load_gather(as_t, [sv])
            adv = plsc.load_gather(ad_t, [dv])
            e = asv + adv
            e = jnp.maximum(e, ALPHA * e) - bv
            w = jnp.exp(e)
            wbuf[pl.ds(i, 16)] = w
            plsc.addupdate_scatter(den, [dv], w)

        dgat.wait()

        @plsc.parallel_loop(0, C, 2, unroll=8, carry=rep8)
        def _(i, iv):
            hv = plsc.load_gather(hrows, [iv, col8])
            wv = plsc.load_gather(wbuf, [iv])
            dv2 = plsc.load_gather(dstb, [iv])
            prod = hv * wv
            plsc.addupdate_scatter(acc, [col8, dv2], prod)
            return iv + 2

    pltpu.sync_copy(acc, acc_o.at[c, h])
    pltpu.sync_copy(den, den_o.at[c, h])


def _sc_compiler_params():
    cp = pltpu.CompilerParams()
    fields = pltpu.CompilerParams.__dataclass_fields__
    if "needs_layout_passes" in fields:
        cp = dataclasses.replace(cp, needs_layout_passes=False)
    if "use_tc_tiling_on_sc" in fields:
        cp = dataclasses.replace(cp, use_tc_tiling_on_sc=False)
    return cp


def _make_edge_pass(NH, NC):
    mesh = plsc.VectorSubcoreMesh(core_axis_name="c", subcore_axis_name="s")
    return pl.kernel(
        functools.partial(_edge_pass_body, NH, NC),
        out_type=[
            jax.ShapeDtypeStruct((NC, NH, NHID, NP), jnp.float32),
            jax.ShapeDtypeStruct((NC, NH, NP), jnp.float32),
        ],
        mesh=mesh,
        scratch_types=[
            pltpu.VMEM((NP,), jnp.float32),
            pltpu.VMEM((NP,), jnp.float32),
            pltpu.VMEM((16,), jnp.float32),
            pltpu.VMEM((NHID, NP), jnp.float32),
            pltpu.VMEM((NP,), jnp.float32),
            pltpu.VMEM((C,), jnp.int32),
            pltpu.VMEM((C,), jnp.int32),
            pltpu.VMEM((C,), jnp.int32),
            pltpu.VMEM((C,), jnp.float32),
            pltpu.VMEM((C, 16), jnp.float32),
            pltpu.SemaphoreType.DMA,
            pltpu.SemaphoreType.DMA,
        ],
        compiler_params=_sc_compiler_params(),
    )


def _prep2_body(acc_ref, den_ref, wl1_ref, bl1_ref, w2_ref, a2s_ref, a2d_ref,
                htab_ref, astab_ref, adtab_ref, btab_ref, mx_ref):
    nb = pl.program_id(0)
    a = jnp.sum(acc_ref[...], axis=0)
    d = jnp.sum(den_ref[...], axis=0)
    zs = []
    for h in range(NHEADS):
        oT = a[h] / (d[h][None, :] + 1e-16)
        oT = _elu(oT)
        zT = _elu(_dgt(wl1_ref[h], oT) + bl1_ref[h])
        zs.append(zT)
    hcatT = jnp.concatenate(zs, axis=0)
    h2T = _dgt(w2_ref[...], hcatT)
    h2 = _dgt(h2T, jnp.eye(NHID, dtype=jnp.float32))
    htab_ref[...] = jnp.concatenate(
        [h2, jnp.zeros((BL, NHID), jnp.float32)], axis=1)
    asv = _dgt(a2s_ref[...], h2T)
    adv = _dgt(a2d_ref[...], h2T)
    astab_ref[...] = asv[None]
    adtab_ref[...] = adv[None]
    m_as = jnp.max(asv)
    m_ad = jnp.max(adv)

    @pl.when(nb == 0)
    def _():
        mx_ref[0] = m_as
        mx_ref[1] = m_ad

    @pl.when(nb != 0)
    def _():
        mx_ref[0] = jnp.maximum(mx_ref[0], m_as)
        mx_ref[1] = jnp.maximum(mx_ref[1], m_ad)

    @pl.when(nb == NBL - 1)
    def _():
        btab_ref[...] = jnp.full((1, 1, 16), _leaky(mx_ref[0] + mx_ref[1]),
                                 jnp.float32)


def _prep2(acc1, den1, Wl1, bl1, W2, a2s, a2d):
    return pl.pallas_call(
        _prep2_body,
        grid=(NBL,),
        in_specs=[
            pl.BlockSpec((8, NHEADS, NHID, BL), lambda nb: (0, 0, 0, nb)),
            pl.BlockSpec((8, NHEADS, BL), lambda nb: (0, 0, nb)),
            pl.BlockSpec((NHEADS, NHID, NHID), lambda nb: (0, 0, 0)),
            pl.BlockSpec((NHEADS, NHID, 1), lambda nb: (0, 0, 0)),
            pl.BlockSpec((NHEADS * NHID, NHID), lambda nb: (0, 0)),
            pl.BlockSpec((NHID, 1), lambda nb: (0, 0)),
            pl.BlockSpec((NHID, 1), lambda nb: (0, 0)),
        ],
        out_specs=[
            pl.BlockSpec((BL, 16), lambda nb: (nb, 0)),
            pl.BlockSpec((1, 1, BL), lambda nb: (0, 0, nb)),
            pl.BlockSpec((1, 1, BL), lambda nb: (0, 0, nb)),
            pl.BlockSpec((1, 1, 16), lambda nb: (0, 0, 0)),
        ],
        out_shape=[
            jax.ShapeDtypeStruct((NP, 16), jnp.float32),
            jax.ShapeDtypeStruct((1, 1, NP), jnp.float32),
            jax.ShapeDtypeStruct((1, 1, NP), jnp.float32),
            jax.ShapeDtypeStruct((1, 1, 16), jnp.float32),
        ],
        scratch_shapes=[pltpu.SMEM((2,), jnp.float32)],
    )(acc1, den1, Wl1, bl1.reshape(NHEADS, NHID, 1), W2,
      a2s.reshape(NHID, 1), a2d.reshape(NHID, 1))


def _readout_body(acc_ref, den_ref, wl2_ref, bl2_ref, wlin1_ref, blin1_ref,
                  wlin2_ref, blin2_ref, out_ref, g_ref):
    nb = pl.program_id(0)
    oT = jnp.sum(acc_ref[:, 0], axis=0)
    d = jnp.sum(den_ref[:, 0], axis=0)
    oT = _elu(oT / (d[None, :] + 1e-16))
    t1 = _elu(_dgt(wl2_ref[...], oT) + bl2_ref[...])
    t1 = _elu(t1)
    t2 = _elu(_dgt(wlin1_ref[...], t1) + blin1_ref[...])
    gidx = nb * BL + lax.broadcasted_iota(jnp.int32, (2 * LIN, BL), 1)
    t2 = jnp.where(gidx < N, t2, 0.0)
    gp = jnp.sum(t2, axis=1, keepdims=True)

    @pl.when(nb == 0)
    def _():
        g_ref[...] = gp

    @pl.when(nb != 0)
    def _():
        g_ref[...] = g_ref[...] + gp

    @pl.when(nb == NBL - 1)
    def _():
        logits = (_dgt(g_ref[...], wlin2_ref[...])
                  + blin2_ref[...])
        m = jnp.max(logits)
        ls = logits - m
        out_ref[...] = ls - jnp.log(jnp.sum(jnp.exp(ls)))


def _readout(acc2, den2, Wl2, bl2, W_lin1, b_lin1, W_lin2, b_lin2):
    return pl.pallas_call(
        _readout_body,
        grid=(NBL,),
        in_specs=[
            pl.BlockSpec((NWORKERS, 1, NHID, BL), lambda nb: (0, 0, 0, nb)),
            pl.BlockSpec((NWORKERS, 1, BL), lambda nb: (0, 0, nb)),
            pl.BlockSpec((NHID, LIN), lambda nb: (0, 0)),
            pl.BlockSpec((LIN, 1), lambda nb: (0, 0)),
            pl.BlockSpec((LIN, 2 * LIN), lambda nb: (0, 0)),
            pl.BlockSpec((2 * LIN, 1), lambda nb: (0, 0)),
            pl.BlockSpec((2 * LIN, NCLASS), lambda nb: (0, 0)),
            pl.BlockSpec((1, NCLASS), lambda nb: (0, 0)),
        ],
        out_specs=[pl.BlockSpec((1, NCLASS), lambda nb: (0, 0))],
        out_shape=[jax.ShapeDtypeStruct((1, NCLASS), jnp.float32)],
        scratch_shapes=[pltpu.VMEM((2 * LIN, 1), jnp.float32)],
    )(acc2, den2, Wl2, bl2, W_lin1, b_lin1, W_lin2, b_lin2)


def kernel(x, edge_index, W1, a_src1, a_dst1, Wl1, bl1, W2, a_src2, a_dst2,
           Wl2, bl2, W_lin1, b_lin1, W_lin2, b_lin2):
    src = edge_index[0]
    dst = edge_index[1]
    xp = jnp.pad(x, ((0, NP - N), (0, 0)))

    htab1, astab1, adtab1, btab1 = _prep1(xp, W1, a_src1, a_dst1)

    l1 = _make_edge_pass(NH=NHEADS, NC=8)
    acc1, den1 = l1(htab1, astab1, adtab1, btab1, src, dst)

    htab2, astab2, adtab2, btab2 = _prep2(acc1, den1, Wl1, bl1, W2,
                                          a_src2, a_dst2)

    l2 = _make_edge_pass(NH=1, NC=NWORKERS)
    acc2, den2 = l2(htab2, astab2, adtab2, btab2, src, dst)

    (out,) = _readout(acc2, den2,
                      Wl2, bl2.reshape(LIN, 1),
                      W_lin1, b_lin1.reshape(2 * LIN, 1),
                      W_lin2, b_lin2.reshape(1, NCLASS))
    return out


# speedup vs baseline: 68.4522x; 1.0028x over previous
import dataclasses
import functools

import jax
import jax.numpy as jnp
from jax import lax
from jax.experimental import pallas as pl
from jax.experimental.pallas import tpu as pltpu
from jax.experimental.pallas import tpu_sc as plsc

N = 10000
NP = 10240
E = 320000
NFEAT = 128
NHID = 8
NHEADS = 4
LIN = 64
NCLASS = 10
ALPHA = 0.2

BL = 1024
NBL = NP // BL
NWORKERS = 32


def _elu(v):
    return jnp.where(v > 0, v, jnp.exp(jnp.minimum(v, 0.0)) - 1.0)


def _leaky(v):
    return jnp.maximum(v, ALPHA * v)


def _dgt(a, b):
    return lax.dot_general(a, b, (((0,), (0,)), ((), ())),
                           preferred_element_type=jnp.float32)


def _prep1_body(x_ref, w1_ref, as1_ref, ad1_ref,
                htab_ref, astab_ref, adtab_ref, btab_ref, mx_ref):
    nb = pl.program_id(1)
    hh = pl.program_id(0)
    xb = x_ref[...]
    hb = jnp.dot(xb, w1_ref[0], preferred_element_type=jnp.float32)
    htab_ref[...] = jnp.concatenate(
        [hb, jnp.zeros((BL, NHID), jnp.float32)], axis=1)
    va_s = jnp.dot(w1_ref[0], as1_ref[0],
                   preferred_element_type=jnp.float32)
    va_d = jnp.dot(w1_ref[0], ad1_ref[0],
                   preferred_element_type=jnp.float32)
    asv = lax.dot_general(va_s, xb, (((0,), (1,)), ((), ())),
                          preferred_element_type=jnp.float32)
    adv = lax.dot_general(va_d, xb, (((0,), (1,)), ((), ())),
                          preferred_element_type=jnp.float32)
    astab_ref[...] = asv[None]
    adtab_ref[...] = adv[None]
    m_as = jnp.max(asv)
    m_ad = jnp.max(adv)

    @pl.when(nb == 0)
    def _():
        mx_ref[0] = m_as
        mx_ref[1] = m_ad

    @pl.when(nb != 0)
    def _():
        mx_ref[0] = jnp.maximum(mx_ref[0], m_as)
        mx_ref[1] = jnp.maximum(mx_ref[1], m_ad)

    @pl.when(nb == NBL - 1)
    def _():
        btab_ref[...] = jnp.full((1, 1, 16), _leaky(mx_ref[0] + mx_ref[1]),
                                 jnp.float32)


def _prep1(xp, W1, a_src1, a_dst1):
    return pl.pallas_call(
        _prep1_body,
        grid=(NHEADS, NBL),
        in_specs=[
            pl.BlockSpec((BL, NFEAT), lambda hh, nb: (nb, 0)),
            pl.BlockSpec((1, NFEAT, NHID), lambda hh, nb: (hh, 0, 0)),
            pl.BlockSpec((1, NHID, 1), lambda hh, nb: (hh, 0, 0)),
            pl.BlockSpec((1, NHID, 1), lambda hh, nb: (hh, 0, 0)),
        ],
        out_specs=[
            pl.BlockSpec((BL, 16), lambda hh, nb: (hh * NBL + nb, 0)),
            pl.BlockSpec((1, 1, BL), lambda hh, nb: (hh, 0, nb)),
            pl.BlockSpec((1, 1, BL), lambda hh, nb: (hh, 0, nb)),
            pl.BlockSpec((1, 1, 16), lambda hh, nb: (hh, 0, 0)),
        ],
        out_shape=[
            jax.ShapeDtypeStruct((NHEADS * NP, 16), jnp.float32),
            jax.ShapeDtypeStruct((NHEADS, 1, NP), jnp.float32),
            jax.ShapeDtypeStruct((NHEADS, 1, NP), jnp.float32),
            jax.ShapeDtypeStruct((NHEADS, 1, 16), jnp.float32),
        ],
        scratch_shapes=[pltpu.SMEM((2,), jnp.float32)],
    )(xp, W1, a_src1.reshape(NHEADS, NHID, 1), a_dst1.reshape(NHEADS, NHID, 1))


C = 800


def _edge_pass_body(NH, NC,
                    htab, astab, adtab, btab, src_h, dst_h,
                    acc_o, den_o,
                    as_t, ad_t, bt, acc, den, srcb, dstb, gidxb, wbuf,
                    hrows, sem0, sem1):
    EPW = E // NC
    NCH = EPW // C
    cid = lax.axis_index("c")
    sid = lax.axis_index("s")
    wid = sid * 2 + cid
    h = wid % NH
    c = wid // NH

    pltpu.sync_copy(astab.at[h, 0], as_t)
    pltpu.sync_copy(adtab.at[h, 0], ad_t)
    pltpu.sync_copy(btab.at[h, 0], bt)

    zf = jnp.zeros((16,), jnp.float32)

    for j in range(NHID):
        @plsc.parallel_loop(0, NP, 16, unroll=8)
        def _(i):
            acc[j, pl.ds(i, 16)] = zf

    @plsc.parallel_loop(0, NP, 16, unroll=8)
    def _(i):
        den[pl.ds(i, 16)] = zf

    bv = bt[...]
    iot = lax.iota(jnp.int32, 16)
    col8 = lax.rem(iot, 8)
    rep8 = lax.div(iot, 8)
    hoff = h * NP

    @pl.loop(0, NCH)
    def _(k):
        base = c * EPW + k * C
        dsrc = pltpu.async_copy(src_h.at[pl.ds(base, C)], srcb, sem0)
        ddst = pltpu.async_copy(dst_h.at[pl.ds(base, C)], dstb, sem1)
        dsrc.wait()

        @plsc.parallel_loop(0, C, 16, unroll=10)
        def _(i):
            gidxb[pl.ds(i, 16)] = srcb[pl.ds(i, 16)] + hoff

        dgat = pltpu.async_copy(htab.at[gidxb], hrows, sem0)
        ddst.wait()

        @plsc.parallel_loop(0, C, 16, unroll=4)
        def _(i):
            sv = srcb[pl.ds(i, 16)]
            dv = dstb[pl.ds(i, 16)]
            asv = plsc.load_gather(as_t, [sv])
            adv = plsc.load_gather(ad_t, [dv])
            e = asv + adv
            e = jnp.maximum(e, ALPHA * e) - bv
            w = jnp.exp(e)
            wbuf[pl.ds(i, 16)] = w
            plsc.addupdate_scatter(den, [dv], w)

        dgat.wait()

        @plsc.parallel_loop(0, C, 2, unroll=8, carry=rep8)
        def _(i, iv):
            hv = plsc.load_gather(hrows, [iv, col8])
            wv = plsc.load_gather(wbuf, [iv])
            dv2 = plsc.load_gather(dstb, [iv])
            prod = hv * wv
            plsc.addupdate_scatter(acc, [col8, dv2], prod)
            return iv + 2

    pltpu.sync_copy(acc, acc_o.at[c, h])
    pltpu.sync_copy(den, den_o.at[c, h])


def _sc_compiler_params():
    cp = pltpu.CompilerParams()
    fields = pltpu.CompilerParams.__dataclass_fields__
    if "needs_layout_passes" in fields:
        cp = dataclasses.replace(cp, needs_layout_passes=False)
    if "use_tc_tiling_on_sc" in fields:
        cp = dataclasses.replace(cp, use_tc_tiling_on_sc=False)
    return cp


def _make_edge_pass(NH, NC):
    mesh = plsc.VectorSubcoreMesh(core_axis_name="c", subcore_axis_name="s")
    return pl.kernel(
        functools.partial(_edge_pass_body, NH, NC),
        out_type=[
            jax.ShapeDtypeStruct((NC, NH, NHID, NP), jnp.float32),
            jax.ShapeDtypeStruct((NC, NH, NP), jnp.float32),
        ],
        mesh=mesh,
        scratch_types=[
            pltpu.VMEM((NP,), jnp.float32),
            pltpu.VMEM((NP,), jnp.float32),
            pltpu.VMEM((16,), jnp.float32),
            pltpu.VMEM((NHID, NP), jnp.float32),
            pltpu.VMEM((NP,), jnp.float32),
            pltpu.VMEM((C,), jnp.int32),
            pltpu.VMEM((C,), jnp.int32),
            pltpu.VMEM((C,), jnp.int32),
            pltpu.VMEM((C,), jnp.float32),
            pltpu.VMEM((C, 16), jnp.float32),
            pltpu.SemaphoreType.DMA,
            pltpu.SemaphoreType.DMA,
        ],
        compiler_params=_sc_compiler_params(),
    )


def _prep2_body(acc_ref, den_ref, wl1_ref, bl1_ref, w2_ref, a2s_ref, a2d_ref,
                htab_ref, astab_ref, adtab_ref, btab_ref, mx_ref):
    nb = pl.program_id(0)
    a = jnp.sum(acc_ref[...], axis=0)
    d = jnp.sum(den_ref[...], axis=0)
    zs = []
    for h in range(NHEADS):
        oT = a[h] / (d[h][None, :] + 1e-16)
        oT = _elu(oT)
        zT = _elu(_dgt(wl1_ref[h], oT) + bl1_ref[h])
        zs.append(zT)
    hcatT = jnp.concatenate(zs, axis=0)
    h2T = _dgt(w2_ref[...], hcatT)
    h2 = _dgt(h2T, jnp.eye(NHID, dtype=jnp.float32))
    htab_ref[...] = jnp.concatenate(
        [h2, jnp.zeros((BL, NHID), jnp.float32)], axis=1)
    asv = _dgt(a2s_ref[...], h2T)
    adv = _dgt(a2d_ref[...], h2T)
    astab_ref[...] = asv[None]
    adtab_ref[...] = adv[None]
    m_as = jnp.max(asv)
    m_ad = jnp.max(adv)

    @pl.when(nb == 0)
    def _():
        mx_ref[0] = m_as
        mx_ref[1] = m_ad

    @pl.when(nb != 0)
    def _():
        mx_ref[0] = jnp.maximum(mx_ref[0], m_as)
        mx_ref[1] = jnp.maximum(mx_ref[1], m_ad)

    @pl.when(nb == NBL - 1)
    def _():
        btab_ref[...] = jnp.full((1, 1, 16), _leaky(mx_ref[0] + mx_ref[1]),
                                 jnp.float32)


def _prep2(acc1, den1, Wl1, bl1, W2, a2s, a2d):
    return pl.pallas_call(
        _prep2_body,
        grid=(NBL,),
        in_specs=[
            pl.BlockSpec((8, NHEADS, NHID, BL), lambda nb: (0, 0, 0, nb)),
            pl.BlockSpec((8, NHEADS, BL), lambda nb: (0, 0, nb)),
            pl.BlockSpec((NHEADS, NHID, NHID), lambda nb: (0, 0, 0)),
            pl.BlockSpec((NHEADS, NHID, 1), lambda nb: (0, 0, 0)),
            pl.BlockSpec((NHEADS * NHID, NHID), lambda nb: (0, 0)),
            pl.BlockSpec((NHID, 1), lambda nb: (0, 0)),
            pl.BlockSpec((NHID, 1), lambda nb: (0, 0)),
        ],
        out_specs=[
            pl.BlockSpec((BL, 16), lambda nb: (nb, 0)),
            pl.BlockSpec((1, 1, BL), lambda nb: (0, 0, nb)),
            pl.BlockSpec((1, 1, BL), lambda nb: (0, 0, nb)),
            pl.BlockSpec((1, 1, 16), lambda nb: (0, 0, 0)),
        ],
        out_shape=[
            jax.ShapeDtypeStruct((NP, 16), jnp.float32),
            jax.ShapeDtypeStruct((1, 1, NP), jnp.float32),
            jax.ShapeDtypeStruct((1, 1, NP), jnp.float32),
            jax.ShapeDtypeStruct((1, 1, 16), jnp.float32),
        ],
        scratch_shapes=[pltpu.SMEM((2,), jnp.float32)],
    )(acc1, den1, Wl1, bl1.reshape(NHEADS, NHID, 1), W2,
      a2s.reshape(NHID, 1), a2d.reshape(NHID, 1))


def _readout_body(acc_ref, den_ref, wl2_ref, bl2_ref, wlin1_ref, blin1_ref,
                  wlin2_ref, blin2_ref, out_ref, g_ref):
    nb = pl.program_id(0)
    oT = jnp.sum(acc_ref[:, 0], axis=0)
    d = jnp.sum(den_ref[:, 0], axis=0)
    oT = _elu(oT / (d[None, :] + 1e-16))
    t1 = _elu(_dgt(wl2_ref[...], oT) + bl2_ref[...])
    t1 = _elu(t1)
    t2 = _elu(_dgt(wlin1_ref[...], t1) + blin1_ref[...])
    gidx = nb * BL + lax.broadcasted_iota(jnp.int32, (2 * LIN, BL), 1)
    t2 = jnp.where(gidx < N, t2, 0.0)
    gp = jnp.sum(t2, axis=1, keepdims=True)

    @pl.when(nb == 0)
    def _():
        g_ref[...] = gp

    @pl.when(nb != 0)
    def _():
        g_ref[...] = g_ref[...] + gp

    @pl.when(nb == NBL - 1)
    def _():
        logits = (_dgt(g_ref[...], wlin2_ref[...])
                  + blin2_ref[...])
        m = jnp.max(logits)
        ls = logits - m
        out_ref[...] = ls - jnp.log(jnp.sum(jnp.exp(ls)))


def _readout(acc2, den2, Wl2, bl2, W_lin1, b_lin1, W_lin2, b_lin2):
    return pl.pallas_call(
        _readout_body,
        grid=(NBL,),
        in_specs=[
            pl.BlockSpec((NWORKERS, 1, NHID, BL), lambda nb: (0, 0, 0, nb)),
            pl.BlockSpec((NWORKERS, 1, BL), lambda nb: (0, 0, nb)),
            pl.BlockSpec((NHID, LIN), lambda nb: (0, 0)),
            pl.BlockSpec((LIN, 1), lambda nb: (0, 0)),
            pl.BlockSpec((LIN, 2 * LIN), lambda nb: (0, 0)),
            pl.BlockSpec((2 * LIN, 1), lambda nb: (0, 0)),
            pl.BlockSpec((2 * LIN, NCLASS), lambda nb: (0, 0)),
            pl.BlockSpec((1, NCLASS), lambda nb: (0, 0)),
        ],
        out_specs=[pl.BlockSpec((1, NCLASS), lambda nb: (0, 0))],
        out_shape=[jax.ShapeDtypeStruct((1, NCLASS), jnp.float32)],
        scratch_shapes=[pltpu.VMEM((2 * LIN, 1), jnp.float32)],
    )(acc2, den2, Wl2, bl2, W_lin1, b_lin1, W_lin2, b_lin2)


def kernel(x, edge_index, W1, a_src1, a_dst1, Wl1, bl1, W2, a_src2, a_dst2,
           Wl2, bl2, W_lin1, b_lin1, W_lin2, b_lin2):
    src = edge_index[0]
    dst = edge_index[1]
    xp = jnp.pad(x, ((0, NP - N), (0, 0)))

    htab1, astab1, adtab1, btab1 = _prep1(xp, W1, a_src1, a_dst1)

    l1 = _make_edge_pass(NH=NHEADS, NC=8)
    acc1, den1 = l1(htab1, astab1, adtab1, btab1, src, dst)

    htab2, astab2, adtab2, btab2 = _prep2(acc1, den1, Wl1, bl1, W2,
                                          a_src2, a_dst2)

    l2 = _make_edge_pass(NH=1, NC=NWORKERS)
    acc2, den2 = l2(htab2, astab2, adtab2, btab2, src, dst)

    (out,) = _readout(acc2, den2,
                      Wl2, bl2.reshape(LIN, 1),
                      W_lin1, b_lin1.reshape(2 * LIN, 1),
                      W_lin2, b_lin2.reshape(1, NCLASS))
    return out
